# baseline (speedup 1.0000x reference)
import functools

import numpy as np
import jax
import jax.numpy as jnp
from jax import lax
from jax.experimental import pallas as pl
from jax.experimental.pallas import tpu as pltpu
from jax.experimental.pallas import tpu_sc as plsc

_PAD_ID = 0
_LANES = 16
_NBUF = 3
_HB = 32


def _pos_encoding(seq_len, d_model):
    pos = np.arange(seq_len, dtype=np.float32)[:, None]
    div = np.exp(
        np.arange(0, d_model, 2, dtype=np.float32) * (-np.log(10000.0) / d_model)
    )
    pe = np.zeros((seq_len, d_model), dtype=np.float32)
    pe[:, 0::2] = np.sin(pos * div)
    pe[:, 1::2] = np.cos(pos * div)
    return pe


@functools.lru_cache(maxsize=None)
def _make_embed_kernel(S, Lp, B, D, NC, NS):
    NW = NC * NS
    assert S % NW == 0 and D % 128 == 0 and B % _HB == 0
    n_pos = S // NW
    CPP = B // _HB
    TJ = CPP * n_pos
    T = 2 * TJ
    mesh = plsc.VectorSubcoreMesh(core_axis_name="c", subcore_axis_name="s")

    @functools.partial(
        pl.kernel,
        mesh=mesh,
        out_type=[
            jax.ShapeDtypeStruct((S, B, D), jnp.float32),
            jax.ShapeDtypeStruct((Lp, B, D), jnp.float32),
        ],
        scratch_types=[
            pltpu.VMEM((2 * n_pos * B,), jnp.int32),
            pltpu.VMEM((_NBUF * _HB, D), jnp.float32),
            pltpu.VMEM(((n_pos + 1) * D,), jnp.float32),
        ]
        + [pltpu.SemaphoreType.DMA] * (2 * _NBUF + 1),
    )
    def k(xt_hbm, yt_hbm, pe_hbm, src_hbm, tgt_hbm, xe_out, ye_out,
          idx_v, rows_v, pe_v, *sems):
        wid = lax.axis_index("s") * NC + lax.axis_index("c")
        gsems = list(sems[:_NBUF])
        ssems = list(sems[_NBUF:2 * _NBUF])
        psem = sems[2 * _NBUF]

        pltpu.sync_copy(xt_hbm.at[pl.ds(wid * B, B)], idx_v.at[pl.ds(0, B)])

        def sched(u):
            vj = u % TJ
            is_ye = u >= TJ
            i = vj // CPP
            h0 = (vj % CPP) * _HB
            pos = wid + NW * i
            return is_ye, i, h0, pos

        def issue_gather(u, b):
            is_ye, i, h0, pos = sched(u)
            if isinstance(is_ye, bool):
                isel = (n_pos * B if is_ye else 0) + i * B + h0
            else:
                isel = lax.select(
                    is_ye, jnp.int32(n_pos * B), jnp.int32(0)) + i * B + h0
            idx_ref = idx_v.at[pl.ds(isel, _HB)]
            dst = rows_v.at[pl.ds(b * _HB, _HB)]

            if isinstance(is_ye, bool):
                tab = tgt_hbm if is_ye else src_hbm
                pltpu.async_copy(tab.at[idx_ref], dst, gsems[b])
            else:
                @pl.when(is_ye)
                def _():
                    pltpu.async_copy(tgt_hbm.at[idx_ref], dst, gsems[b])

                @pl.when(jnp.logical_not(is_ye))
                def _():
                    pltpu.async_copy(src_hbm.at[idx_ref], dst, gsems[b])

        def wait_gather(b):
            pltpu.make_async_copy(
                src_hbm.at[pl.ds(0, _HB)], rows_v.at[pl.ds(b * _HB, _HB)], gsems[b]
            ).wait()

        def issue_scatter(u, b):
            is_ye, i, h0, pos = sched(u)
            srcb = rows_v.at[pl.ds(b * _HB, _HB)]

            def to_ye():
                p = lax.min(pos, Lp - 1)
                pltpu.async_copy(srcb, ye_out.at[p, pl.ds(h0, _HB)], ssems[b])

            def to_xe():
                pltpu.async_copy(srcb, xe_out.at[pos, pl.ds(h0, _HB)], ssems[b])

            if isinstance(is_ye, bool):
                to_ye() if is_ye else to_xe()
            else:
                pl.when(is_ye)(to_ye)
                pl.when(jnp.logical_not(is_ye))(to_xe)

        def wait_scatter(b):
            pltpu.make_async_copy(
                rows_v.at[pl.ds(b * _HB, _HB)], xe_out.at[0, pl.ds(0, _HB)],
                ssems[b]
            ).wait()

        def compute(u, b):
            is_ye, i, h0, pos = sched(u)
            if isinstance(is_ye, bool) and not is_ye:
                pb = i * D
            else:
                clamped = jnp.logical_and(jnp.asarray(is_ye), pos >= Lp)
                pb = lax.select(clamped, jnp.int32(n_pos * D),
                                jnp.int32(1) * (i * D))
            for g in range(D // (16 * _LANES)):
                pe_regs = [
                    pe_v[pl.ds(pb + g * 16 * _LANES + kk * _LANES, _LANES)]
                    for kk in range(16)
                ]

                def body(r, carry):
                    for kk in range(16):
                        sl = pl.ds(g * 16 * _LANES + kk * _LANES, _LANES)
                        row = b * _HB + r
                        rows_v[row, sl] = rows_v[row, sl] + pe_regs[kk]
                    return carry

                lax.fori_loop(0, _HB, body, 0)

        def do_step(u, b):
            v = u + _NBUF - 1
            bg = (b + _NBUF - 1) % _NBUF
            if isinstance(v, int):
                if v < T:
                    if v >= _NBUF:
                        wait_scatter(bg)
                    issue_gather(v, bg)
            else:
                @pl.when(v < T)
                def _():
                    @pl.when(v >= _NBUF)
                    def _():
                        wait_scatter(bg)

                    issue_gather(v, bg)

            wait_gather(b)
            compute(u, b)
            issue_scatter(u, b)

        for b in range(_NBUF - 1):
            issue_gather(b, b)
        cps = []
        for i in range(n_pos):
            p_x = wid + NW * i
            p_y = lax.min(p_x, Lp - 1)
            if i > 0:
                cps.append(pltpu.async_copy(
                    xt_hbm.at[pl.ds(p_x * B, B)], idx_v.at[pl.ds(i * B, B)],
                    psem))
            cps.append(pltpu.async_copy(
                yt_hbm.at[pl.ds(p_y * B, B)], idx_v.at[pl.ds((n_pos + i) * B, B)],
                psem))
            cps.append(pltpu.async_copy(
                pe_hbm.at[pl.ds(p_x * D, D)], pe_v.at[pl.ds(i * D, D)], psem))
        cps.append(pltpu.async_copy(
            pe_hbm.at[pl.ds((Lp - 1) * D, D)], pe_v.at[pl.ds(n_pos * D, D)], psem))
        for c in cps:
            c.wait()

        T_floor = (T // _NBUF) * _NBUF

        def lbody(t, carry):
            for j in range(_NBUF):
                do_step(t + j, j)
            return carry

        lax.fori_loop(0, T_floor // _NBUF, lambda i, c: lbody(i * _NBUF, c), 0)
        for u in range(T_floor, T):
            do_step(u, u % _NBUF)
        for u in range(T - _NBUF, T):
            wait_scatter(u % _NBUF)

    return k


def kernel(x, y, emb_src, emb_tgt):
    B, S = x.shape
    D = emb_src.shape[1]
    tgt = y[:, :-1]
    labels = y[:, 1:]
    Lp = tgt.shape[1]

    pe = jnp.asarray(_pos_encoding(S, D)).reshape(-1)
    xt = x.T.reshape(-1)
    yt = tgt.T.reshape(-1)

    info = plsc.get_sparse_core_info()
    embed = _make_embed_kernel(S, Lp, B, D, info.num_cores, info.num_subcores)
    xe, ye = embed(xt, yt, pe, emb_src, emb_tgt)

    src_padding_mask = x == _PAD_ID
    tgt_attention_mask = jnp.where(
        jnp.triu(jnp.ones((Lp, Lp), dtype=bool), k=1), -jnp.inf, 0.0
    ).astype(jnp.float32)
    tgt_padding_mask = tgt == _PAD_ID
    return (
        xe,
        src_padding_mask,
        src_padding_mask,
        ye,
        tgt_attention_mask,
        tgt_padding_mask,
        labels,
    )


# speedup vs baseline: 1.1031x; 1.1031x over previous
import functools

import numpy as np
import jax
import jax.numpy as jnp
from jax import lax
from jax.experimental import pallas as pl
from jax.experimental.pallas import tpu as pltpu
from jax.experimental.pallas import tpu_sc as plsc

_PAD_ID = 0
_LANES = 16
_NBUF = 3
_HB = 32


def _pos_encoding(seq_len, d_model):
    pos = np.arange(seq_len, dtype=np.float32)[:, None]
    div = np.exp(
        np.arange(0, d_model, 2, dtype=np.float32) * (-np.log(10000.0) / d_model)
    )
    pe = np.zeros((seq_len, d_model), dtype=np.float32)
    pe[:, 0::2] = np.sin(pos * div)
    pe[:, 1::2] = np.cos(pos * div)
    return pe


@functools.lru_cache(maxsize=None)
def _make_embed_kernel(S, Lp, B, D, NC, NS):
    NW = NC * NS
    assert S % NW == 0 and D % 128 == 0 and B % _HB == 0
    n_pos = S // NW
    CPP = B // _HB
    TJ = CPP * n_pos
    T = 2 * TJ
    mesh = plsc.VectorSubcoreMesh(core_axis_name="c", subcore_axis_name="s")

    @functools.partial(
        pl.kernel,
        mesh=mesh,
        out_type=[
            jax.ShapeDtypeStruct((S, B, D), jnp.float32),
            jax.ShapeDtypeStruct((Lp, B, D), jnp.float32),
        ],
        scratch_types=[
            pltpu.VMEM((2 * n_pos * B,), jnp.int32),
            pltpu.VMEM((_NBUF * _HB, D), jnp.float32),
            pltpu.VMEM(((n_pos + 1) * D,), jnp.float32),
        ]
        + [pltpu.SemaphoreType.DMA] * (2 * _NBUF + 1),
    )
    def k(xt_hbm, yt_hbm, pe_hbm, src_hbm, tgt_hbm, xe_out, ye_out,
          idx_v, rows_v, pe_v, *sems):
        wid = lax.axis_index("s") * NC + lax.axis_index("c")
        gsems = list(sems[:_NBUF])
        ssems = list(sems[_NBUF:2 * _NBUF])
        psem = sems[2 * _NBUF]

        pltpu.sync_copy(xt_hbm.at[pl.ds(wid * B, B)], idx_v.at[pl.ds(0, B)])

        def sched(u):
            vj = u % TJ
            is_ye = u >= TJ
            i = vj // CPP
            h0 = (vj % CPP) * _HB
            pos = wid + NW * i
            return is_ye, i, h0, pos

        def issue_gather(u, b):
            is_ye, i, h0, pos = sched(u)
            if isinstance(is_ye, bool):
                isel = (n_pos * B if is_ye else 0) + i * B + h0
            else:
                isel = lax.select(
                    is_ye, jnp.int32(n_pos * B), jnp.int32(0)) + i * B + h0
            idx_ref = idx_v.at[pl.ds(isel, _HB)]
            dst = rows_v.at[pl.ds(b * _HB, _HB)]

            if isinstance(is_ye, bool):
                tab = tgt_hbm if is_ye else src_hbm
                pltpu.async_copy(tab.at[idx_ref], dst, gsems[b])
            else:
                @pl.when(is_ye)
                def _():
                    pltpu.async_copy(tgt_hbm.at[idx_ref], dst, gsems[b])

                @pl.when(jnp.logical_not(is_ye))
                def _():
                    pltpu.async_copy(src_hbm.at[idx_ref], dst, gsems[b])

        def wait_gather(b):
            pltpu.make_async_copy(
                src_hbm.at[pl.ds(0, _HB)], rows_v.at[pl.ds(b * _HB, _HB)], gsems[b]
            ).wait()

        def issue_scatter(u, b):
            is_ye, i, h0, pos = sched(u)
            srcb = rows_v.at[pl.ds(b * _HB, _HB)]

            def to_ye():
                p = lax.min(pos, Lp - 1)
                pltpu.async_copy(srcb, ye_out.at[p, pl.ds(h0, _HB)], ssems[b])

            def to_xe():
                pltpu.async_copy(srcb, xe_out.at[pos, pl.ds(h0, _HB)], ssems[b])

            if isinstance(is_ye, bool):
                to_ye() if is_ye else to_xe()
            else:
                pl.when(is_ye)(to_ye)
                pl.when(jnp.logical_not(is_ye))(to_xe)

        def wait_scatter(b):
            pltpu.make_async_copy(
                rows_v.at[pl.ds(b * _HB, _HB)], xe_out.at[0, pl.ds(0, _HB)],
                ssems[b]
            ).wait()

        def compute(u, b):
            is_ye, i, h0, pos = sched(u)
            if isinstance(is_ye, bool) and not is_ye:
                pb = i * D
            else:
                clamped = jnp.logical_and(jnp.asarray(is_ye), pos >= Lp)
                pb = lax.select(clamped, jnp.int32(n_pos * D),
                                jnp.int32(1) * (i * D))
            for g in range(D // (8 * _LANES)):
                pe_regs = [
                    pe_v[pl.ds(pb + g * 8 * _LANES + kk * _LANES, _LANES)]
                    for kk in range(8)
                ]

                def body(r, carry):
                    for kk in range(8):
                        sl = pl.ds(g * 8 * _LANES + kk * _LANES, _LANES)
                        row = b * _HB + r
                        rows_v[row, sl] = rows_v[row, sl] + pe_regs[kk]
                    return carry

                lax.fori_loop(0, _HB, body, 0)

        def do_step(u, b):
            v = u + _NBUF - 1
            bg = (b + _NBUF - 1) % _NBUF
            if isinstance(v, int):
                if v < T:
                    if v >= _NBUF:
                        wait_scatter(bg)
                    issue_gather(v, bg)
            else:
                @pl.when(v < T)
                def _():
                    @pl.when(v >= _NBUF)
                    def _():
                        wait_scatter(bg)

                    issue_gather(v, bg)

            wait_gather(b)
            compute(u, b)
            issue_scatter(u, b)

        for b in range(_NBUF - 1):
            issue_gather(b, b)
        cps = []
        for i in range(n_pos):
            p_x = wid + NW * i
            p_y = lax.min(p_x, Lp - 1)
            if i > 0:
                cps.append(pltpu.async_copy(
                    xt_hbm.at[pl.ds(p_x * B, B)], idx_v.at[pl.ds(i * B, B)],
                    psem))
            cps.append(pltpu.async_copy(
                yt_hbm.at[pl.ds(p_y * B, B)], idx_v.at[pl.ds((n_pos + i) * B, B)],
                psem))
            cps.append(pltpu.async_copy(
                pe_hbm.at[pl.ds(p_x * D, D)], pe_v.at[pl.ds(i * D, D)], psem))
        cps.append(pltpu.async_copy(
            pe_hbm.at[pl.ds((Lp - 1) * D, D)], pe_v.at[pl.ds(n_pos * D, D)], psem))
        for c in cps:
            c.wait()

        T_floor = (T // _NBUF) * _NBUF

        def lbody(t, carry):
            for j in range(_NBUF):
                do_step(t + j, j)
            return carry

        lax.fori_loop(0, T_floor // _NBUF, lambda i, c: lbody(i * _NBUF, c), 0)
        for u in range(T_floor, T):
            do_step(u, u % _NBUF)
        for u in range(T - _NBUF, T):
            wait_scatter(u % _NBUF)

    return k


def kernel(x, y, emb_src, emb_tgt):
    B, S = x.shape
    D = emb_src.shape[1]
    tgt = y[:, :-1]
    labels = y[:, 1:]
    Lp = tgt.shape[1]

    pe = jnp.asarray(_pos_encoding(S, D)).reshape(-1)
    xt = x.T.reshape(-1)
    yt = tgt.T.reshape(-1)

    info = plsc.get_sparse_core_info()
    embed = _make_embed_kernel(S, Lp, B, D, info.num_cores, info.num_subcores)
    xe, ye = embed(xt, yt, pe, emb_src, emb_tgt)

    src_padding_mask = x == _PAD_ID
    tgt_attention_mask = jnp.where(
        jnp.triu(jnp.ones((Lp, Lp), dtype=bool), k=1), -jnp.inf, 0.0
    ).astype(jnp.float32)
    tgt_padding_mask = tgt == _PAD_ID
    return (
        xe,
        src_padding_mask,
        src_padding_mask,
        ye,
        tgt_attention_mask,
        tgt_padding_mask,
        labels,
    )
